# baseline (speedup 1.0000x reference)
import functools

import jax
import jax.numpy as jnp
from jax.experimental import pallas as pl
from jax.experimental.pallas import tpu as pltpu


def _dis_kernel(det_ref, pct_ref, dis_ref):
    px = pct_ref[0, 0:1, :]
    py = pct_ref[0, 1:2, :]
    pz = pct_ref[0, 2:3, :]
    qx = det_ref[0, :, 0:1]
    qy = det_ref[0, :, 1:2]
    qz = det_ref[0, :, 2:3]
    dx = qx - px
    dy = qy - py
    dz = qz - pz
    d2 = dx * dx + dy * dy + dz * dz
    dis_ref[0] = jnp.sqrt(jnp.min(d2, axis=1, keepdims=True))


def _rank_kernel(n_total, n_close, rb2,
                 discol_ref, disrow_ref, ic_ref):
    r = pl.program_id(1)
    dcol = discol_ref[0]
    drow = disrow_ref[0]
    icol = jax.lax.broadcasted_iota(jnp.int32, (rb2, 1), 0) + r * rb2
    irow = jax.lax.broadcasted_iota(jnp.int32, (1, n_total), 1)
    lt = (drow < dcol) | ((drow == dcol) & (irow < icol))
    rank = jnp.sum(lt.astype(jnp.int32), axis=1, keepdims=True)
    ic_ref[0] = (rank < n_close).astype(jnp.int32)


def _main_kernel(n_np, k_close, k_far, n_bs_iters,
                 det_ref, pct_ref, f5_ref, ic_ref,
                 Wc_ref, bc_ref, W1c_ref, b1c_ref, W2c_ref, b2c_ref,
                 Wf_ref, bf_ref, W1f_ref, b1f_ref, W2f_ref, b2f_ref,
                 out_ref, d2i_s):
    px = pct_ref[0, 0:1, :]
    py = pct_ref[0, 1:2, :]
    pz = pct_ref[0, 2:3, :]
    qx = det_ref[0, :, 0:1]
    qy = det_ref[0, :, 1:2]
    qz = det_ref[0, :, 2:3]
    dx = qx - px
    dy = qy - py
    dz = qz - pz
    d2 = dx * dx + dy * dy + dz * dz
    d2i = jax.lax.bitcast_convert_type(d2, jnp.int32)
    d2i_s[...] = d2i
    dminb = jnp.min(d2i, axis=1, keepdims=True)
    dmaxb = jnp.max(d2i, axis=1, keepdims=True)

    ic = ic_ref[0] != 0
    kk = jnp.where(ic, k_close, k_far)

    def bs(t, c):
        lo, hi = c
        m = lo + ((hi - lo) >> 1)
        d = d2i_s[...]
        cnt = jnp.sum((d <= m).astype(jnp.int32), axis=1, keepdims=True)
        ge = cnt >= kk
        return jnp.where(ge, lo, m + 1), jnp.where(ge, m, hi)

    _, tk = jax.lax.fori_loop(0, n_bs_iters, bs, (dminb, dmaxb))

    wfull = 1.0 / (jnp.sqrt(jnp.maximum(d2, 1e-10)) + 1e-8)
    w = jnp.where(d2i <= tk, wfull, 0.0)
    wsum = jnp.sum(w, axis=1, keepdims=True)

    f5 = f5_ref[0]
    interp = jax.lax.dot(w, f5, precision=jax.lax.Precision.HIGHEST)
    interp = interp * (1.0 / wsum)

    df = jax.lax.Precision.HIGHEST
    hc = jnp.maximum(jax.lax.dot(interp, Wc_ref[...], precision=df) + bc_ref[...], 0.0)
    hc = jnp.maximum(jax.lax.dot(hc, W1c_ref[...], precision=df) + b1c_ref[...], 0.0)
    lc = jnp.tanh(jax.lax.dot(hc, W2c_ref[...], precision=df) + b2c_ref[...])

    hf = jnp.maximum(jax.lax.dot(interp, Wf_ref[...], precision=df) + bf_ref[...], 0.0)
    hf = jnp.maximum(jax.lax.dot(hf, W1f_ref[...], precision=df) + b1f_ref[...], 0.0)
    lf = jnp.tanh(jax.lax.dot(hf, W2f_ref[...], precision=df) + b2f_ref[...])

    out_ref[0] = jnp.where(ic, lc[:, 0:2], lf[:, 0:2])


def kernel(point_cloud, detect_point, feature4, feature5, feature6,
           fps_idx1, fps_idx2,
           W_ind_c, b_ind_c, W1_c, b1_c, W2_c, b2_c,
           W_ind_f, b_ind_f, W1_f, b1_f, W2_f, b2_f):
    B, N, _ = detect_point.shape
    Np = point_cloud.shape[1]
    C = feature5.shape[2]
    n_close = N * 2 // 3
    k_close, k_far = 24, 12
    RB0 = 512
    RB = 256
    RB2 = 512

    pct = jnp.transpose(point_cloud, (0, 2, 1))
    b2 = lambda v: v.reshape(1, -1)

    dis = pl.pallas_call(
        _dis_kernel,
        grid=(B, N // RB0),
        in_specs=[
            pl.BlockSpec((1, RB0, 3), lambda b, r: (b, r, 0)),
            pl.BlockSpec((1, 3, Np), lambda b, r: (b, 0, 0)),
        ],
        out_specs=pl.BlockSpec((1, RB0, 1), lambda b, r: (b, r, 0)),
        out_shape=jax.ShapeDtypeStruct((B, N, 1), jnp.float32),
    )(detect_point, pct)

    is_close = pl.pallas_call(
        functools.partial(_rank_kernel, N, n_close, RB2),
        grid=(B, N // RB2),
        in_specs=[
            pl.BlockSpec((1, RB2, 1), lambda b, r: (b, r, 0)),
            pl.BlockSpec((1, 1, N), lambda b, r: (b, 0, 0)),
        ],
        out_specs=pl.BlockSpec((1, RB2, 1), lambda b, r: (b, r, 0)),
        out_shape=jax.ShapeDtypeStruct((B, N, 1), jnp.int32),
    )(dis, dis.reshape(B, 1, N))

    full = lambda s: pl.BlockSpec(s, lambda b, r: (0, 0))
    logit = pl.pallas_call(
        functools.partial(_main_kernel, Np, k_close, k_far, 20),
        grid=(B, N // RB),
        in_specs=[
            pl.BlockSpec((1, RB, 3), lambda b, r: (b, r, 0)),
            pl.BlockSpec((1, 3, Np), lambda b, r: (b, 0, 0)),
            pl.BlockSpec((1, Np, C), lambda b, r: (b, 0, 0)),
            pl.BlockSpec((1, RB, 1), lambda b, r: (b, r, 0)),
            full(W_ind_c.shape), full((1, b_ind_c.shape[0])),
            full(W1_c.shape), full((1, b1_c.shape[0])),
            full(W2_c.shape), full((1, b2_c.shape[0])),
            full(W_ind_f.shape), full((1, b_ind_f.shape[0])),
            full(W1_f.shape), full((1, b1_f.shape[0])),
            full(W2_f.shape), full((1, b2_f.shape[0])),
        ],
        out_specs=pl.BlockSpec((1, RB, 2), lambda b, r: (b, r, 0)),
        out_shape=jax.ShapeDtypeStruct((B, N, 2), jnp.float32),
        scratch_shapes=[
            pltpu.VMEM((RB, Np), jnp.int32),
        ],
    )(detect_point, pct, feature5, is_close,
      W_ind_c, b2(b_ind_c), W1_c, b2(b1_c), W2_c, b2(b2_c),
      W_ind_f, b2(b_ind_f), W1_f, b2(b1_f), W2_f, b2(b2_f))
    return logit


# speedup vs baseline: 1.1004x; 1.1004x over previous
import functools

import jax
import jax.numpy as jnp
from jax.experimental import pallas as pl
from jax.experimental.pallas import tpu as pltpu


def _dot3(a, b):
    ahi = a.astype(jnp.bfloat16)
    alo = (a - ahi.astype(jnp.float32)).astype(jnp.bfloat16)
    bhi = b.astype(jnp.bfloat16)
    blo = (b - bhi.astype(jnp.float32)).astype(jnp.bfloat16)
    d = functools.partial(jax.lax.dot, preferred_element_type=jnp.float32)
    return d(ahi, bhi) + d(ahi, blo) + d(alo, bhi)


def _dis_kernel(det_ref, pct_ref, dis_ref):
    px = pct_ref[0, 0:1, :]
    py = pct_ref[0, 1:2, :]
    pz = pct_ref[0, 2:3, :]
    qx = det_ref[0, :, 0:1]
    qy = det_ref[0, :, 1:2]
    qz = det_ref[0, :, 2:3]
    dx = qx - px
    dy = qy - py
    dz = qz - pz
    d2 = dx * dx + dy * dy + dz * dz
    dis_ref[0] = jnp.sqrt(jnp.min(d2, axis=1, keepdims=True))


def _rank_kernel(n_total, n_close, rb2,
                 discol_ref, disrow_ref, ic_ref):
    r = pl.program_id(1)
    dcol = discol_ref[0]
    drow = disrow_ref[0]
    icol = jax.lax.broadcasted_iota(jnp.int32, (rb2, 1), 0) + r * rb2
    irow = jax.lax.broadcasted_iota(jnp.int32, (1, n_total), 1)
    lt = (drow < dcol) | ((drow == dcol) & (irow < icol))
    rank = jnp.sum(lt.astype(jnp.int32), axis=1, keepdims=True)
    ic_ref[0] = (rank < n_close).astype(jnp.int32)


def _main_kernel(n_np, k_close, k_far, n_bs_iters,
                 det_ref, pct_ref, f5_ref, ic_ref,
                 Wc_ref, bc_ref, W1c_ref, b1c_ref, W2c_ref, b2c_ref,
                 Wf_ref, bf_ref, W1f_ref, b1f_ref, W2f_ref, b2f_ref,
                 out_ref, d2i_s):
    px = pct_ref[0, 0:1, :]
    py = pct_ref[0, 1:2, :]
    pz = pct_ref[0, 2:3, :]
    qx = det_ref[0, :, 0:1]
    qy = det_ref[0, :, 1:2]
    qz = det_ref[0, :, 2:3]
    dx = qx - px
    dy = qy - py
    dz = qz - pz
    d2 = dx * dx + dy * dy + dz * dz
    d2i = jax.lax.bitcast_convert_type(d2, jnp.int32)
    d2i_s[...] = d2i
    dminb = jnp.min(d2i, axis=1, keepdims=True)
    dmaxb = jnp.max(d2i, axis=1, keepdims=True)

    ic = ic_ref[0] != 0
    kk = jnp.where(ic, k_close, k_far)

    def bs(t, c):
        lo, hi = c
        m = lo + ((hi - lo) >> 1)
        d = d2i_s[...]
        cnt = jnp.sum((d <= m).astype(jnp.int32), axis=1, keepdims=True)
        ge = cnt >= kk
        return jnp.where(ge, lo, m + 1), jnp.where(ge, m, hi)

    _, tk = jax.lax.fori_loop(0, n_bs_iters, bs, (dminb, dmaxb))

    wfull = 1.0 / (jnp.sqrt(jnp.maximum(d2, 1e-10)) + 1e-8)
    w = jnp.where(d2i <= tk, wfull, 0.0)
    wsum = jnp.sum(w, axis=1, keepdims=True)

    f5 = f5_ref[0]
    interp = _dot3(w, f5)
    interp = interp * (1.0 / wsum)

    hc = jnp.maximum(_dot3(interp, Wc_ref[...]) + bc_ref[...], 0.0)
    hc = jnp.maximum(_dot3(hc, W1c_ref[...]) + b1c_ref[...], 0.0)
    lc = jnp.tanh(_dot3(hc, W2c_ref[...]) + b2c_ref[...])

    hf = jnp.maximum(_dot3(interp, Wf_ref[...]) + bf_ref[...], 0.0)
    hf = jnp.maximum(_dot3(hf, W1f_ref[...]) + b1f_ref[...], 0.0)
    lf = jnp.tanh(_dot3(hf, W2f_ref[...]) + b2f_ref[...])

    out_ref[0] = jnp.where(ic, lc[:, 0:2], lf[:, 0:2])


def kernel(point_cloud, detect_point, feature4, feature5, feature6,
           fps_idx1, fps_idx2,
           W_ind_c, b_ind_c, W1_c, b1_c, W2_c, b2_c,
           W_ind_f, b_ind_f, W1_f, b1_f, W2_f, b2_f):
    B, N, _ = detect_point.shape
    Np = point_cloud.shape[1]
    C = feature5.shape[2]
    n_close = N * 2 // 3
    k_close, k_far = 24, 12
    RB0 = 512
    RB = 256
    RB2 = 512

    pct = jnp.transpose(point_cloud, (0, 2, 1))
    b2 = lambda v: v.reshape(1, -1)

    dis = pl.pallas_call(
        _dis_kernel,
        grid=(B, N // RB0),
        in_specs=[
            pl.BlockSpec((1, RB0, 3), lambda b, r: (b, r, 0)),
            pl.BlockSpec((1, 3, Np), lambda b, r: (b, 0, 0)),
        ],
        out_specs=pl.BlockSpec((1, RB0, 1), lambda b, r: (b, r, 0)),
        out_shape=jax.ShapeDtypeStruct((B, N, 1), jnp.float32),
    )(detect_point, pct)

    is_close = pl.pallas_call(
        functools.partial(_rank_kernel, N, n_close, RB2),
        grid=(B, N // RB2),
        in_specs=[
            pl.BlockSpec((1, RB2, 1), lambda b, r: (b, r, 0)),
            pl.BlockSpec((1, 1, N), lambda b, r: (b, 0, 0)),
        ],
        out_specs=pl.BlockSpec((1, RB2, 1), lambda b, r: (b, r, 0)),
        out_shape=jax.ShapeDtypeStruct((B, N, 1), jnp.int32),
    )(dis, dis.reshape(B, 1, N))

    full = lambda s: pl.BlockSpec(s, lambda b, r: (0, 0))
    logit = pl.pallas_call(
        functools.partial(_main_kernel, Np, k_close, k_far, 22),
        grid=(B, N // RB),
        in_specs=[
            pl.BlockSpec((1, RB, 3), lambda b, r: (b, r, 0)),
            pl.BlockSpec((1, 3, Np), lambda b, r: (b, 0, 0)),
            pl.BlockSpec((1, Np, C), lambda b, r: (b, 0, 0)),
            pl.BlockSpec((1, RB, 1), lambda b, r: (b, r, 0)),
            full(W_ind_c.shape), full((1, b_ind_c.shape[0])),
            full(W1_c.shape), full((1, b1_c.shape[0])),
            full(W2_c.shape), full((1, b2_c.shape[0])),
            full(W_ind_f.shape), full((1, b_ind_f.shape[0])),
            full(W1_f.shape), full((1, b1_f.shape[0])),
            full(W2_f.shape), full((1, b2_f.shape[0])),
        ],
        out_specs=pl.BlockSpec((1, RB, 2), lambda b, r: (b, r, 0)),
        out_shape=jax.ShapeDtypeStruct((B, N, 2), jnp.float32),
        scratch_shapes=[
            pltpu.VMEM((RB, Np), jnp.int32),
        ],
    )(detect_point, pct, feature5, is_close,
      W_ind_c, b2(b_ind_c), W1_c, b2(b1_c), W2_c, b2(b2_c),
      W_ind_f, b2(b_ind_f), W1_f, b2(b1_f), W2_f, b2(b2_f))
    return logit


# speedup vs baseline: 1.1367x; 1.0329x over previous
import functools

import jax
import jax.numpy as jnp
from jax.experimental import pallas as pl
from jax.experimental.pallas import tpu as pltpu


def _dot3(a, b):
    ahi = a.astype(jnp.bfloat16)
    alo = (a - ahi.astype(jnp.float32)).astype(jnp.bfloat16)
    bhi = b.astype(jnp.bfloat16)
    blo = (b - bhi.astype(jnp.float32)).astype(jnp.bfloat16)
    d = functools.partial(jax.lax.dot, preferred_element_type=jnp.float32)
    return d(ahi, bhi) + d(ahi, blo) + d(alo, bhi)


def _dis_kernel(det_ref, pct_ref, dis_ref):
    px = pct_ref[0, 0:1, :]
    py = pct_ref[0, 1:2, :]
    pz = pct_ref[0, 2:3, :]
    qx = det_ref[0, :, 0:1]
    qy = det_ref[0, :, 1:2]
    qz = det_ref[0, :, 2:3]
    dx = qx - px
    dy = qy - py
    dz = qz - pz
    d2 = dx * dx + dy * dy + dz * dz
    dis_ref[0] = jnp.sqrt(jnp.min(d2, axis=1, keepdims=True))


def _rank_kernel(n_total, n_close, rb2,
                 discol_ref, disrow_ref, ic_ref):
    r = pl.program_id(1)
    dcol = discol_ref[0]
    drow = disrow_ref[0]
    icol = jax.lax.broadcasted_iota(jnp.int32, (rb2, 1), 0) + r * rb2
    irow = jax.lax.broadcasted_iota(jnp.int32, (1, n_total), 1)
    lt = (drow < dcol) | ((drow == dcol) & (irow < icol))
    rank = jnp.sum(lt.astype(jnp.int32), axis=1, keepdims=True)
    ic_ref[0] = (rank < n_close).astype(jnp.int32)


def _main_kernel(n_np, k_close, k_far, n_bs_iters,
                 det_ref, pct_ref, f5_ref, ic_ref,
                 Wc_ref, bc_ref, W1c_ref, b1c_ref, W2c_ref, b2c_ref,
                 Wf_ref, bf_ref, W1f_ref, b1f_ref, W2f_ref, b2f_ref,
                 out_ref, d2i_s):
    px = pct_ref[0, 0:1, :]
    py = pct_ref[0, 1:2, :]
    pz = pct_ref[0, 2:3, :]
    qx = det_ref[0, :, 0:1]
    qy = det_ref[0, :, 1:2]
    qz = det_ref[0, :, 2:3]
    dx = qx - px
    dy = qy - py
    dz = qz - pz
    d2 = dx * dx + dy * dy + dz * dz
    d2i = jax.lax.bitcast_convert_type(d2, jnp.int32)
    d2i_s[...] = d2i
    dminb = jnp.min(d2i, axis=1, keepdims=True)
    dmaxb = jnp.max(d2i, axis=1, keepdims=True)

    ic = ic_ref[0] != 0
    kk = jnp.where(ic, k_close, k_far)

    kk_gt = n_np - kk

    def bs(t, c):
        lo, hi = c
        m = lo + ((hi - lo) >> 1)
        d = d2i_s[...]
        cnt_gt = jnp.sum(jax.lax.shift_right_logical(m - d, 31), axis=1,
                         keepdims=True)
        ge = cnt_gt <= kk_gt
        return jnp.where(ge, lo, m + 1), jnp.where(ge, m, hi)

    _, tk = jax.lax.fori_loop(0, n_bs_iters, bs, (dminb, dmaxb))

    wfull = jax.lax.rsqrt(jnp.maximum(d2, 1e-10))
    w = jnp.where(d2i <= tk, wfull, 0.0)
    wsum = jnp.sum(w, axis=1, keepdims=True)

    f5 = f5_ref[0]
    interp = _dot3(w, f5)
    interp = interp * (1.0 / wsum)

    hc = jnp.maximum(_dot3(interp, Wc_ref[...]) + bc_ref[...], 0.0)
    hc = jnp.maximum(_dot3(hc, W1c_ref[...]) + b1c_ref[...], 0.0)
    lc = jnp.tanh(_dot3(hc, W2c_ref[...]) + b2c_ref[...])

    hf = jnp.maximum(_dot3(interp, Wf_ref[...]) + bf_ref[...], 0.0)
    hf = jnp.maximum(_dot3(hf, W1f_ref[...]) + b1f_ref[...], 0.0)
    lf = jnp.tanh(_dot3(hf, W2f_ref[...]) + b2f_ref[...])

    out_ref[0] = jnp.where(ic, lc[:, 0:2], lf[:, 0:2])


def kernel(point_cloud, detect_point, feature4, feature5, feature6,
           fps_idx1, fps_idx2,
           W_ind_c, b_ind_c, W1_c, b1_c, W2_c, b2_c,
           W_ind_f, b_ind_f, W1_f, b1_f, W2_f, b2_f):
    B, N, _ = detect_point.shape
    Np = point_cloud.shape[1]
    C = feature5.shape[2]
    n_close = N * 2 // 3
    k_close, k_far = 24, 12
    RB0 = 512
    RB = 256
    RB2 = 512

    pct = jnp.transpose(point_cloud, (0, 2, 1))
    b2 = lambda v: v.reshape(1, -1)

    dis = pl.pallas_call(
        _dis_kernel,
        grid=(B, N // RB0),
        in_specs=[
            pl.BlockSpec((1, RB0, 3), lambda b, r: (b, r, 0)),
            pl.BlockSpec((1, 3, Np), lambda b, r: (b, 0, 0)),
        ],
        out_specs=pl.BlockSpec((1, RB0, 1), lambda b, r: (b, r, 0)),
        out_shape=jax.ShapeDtypeStruct((B, N, 1), jnp.float32),
    )(detect_point, pct)

    is_close = pl.pallas_call(
        functools.partial(_rank_kernel, N, n_close, RB2),
        grid=(B, N // RB2),
        in_specs=[
            pl.BlockSpec((1, RB2, 1), lambda b, r: (b, r, 0)),
            pl.BlockSpec((1, 1, N), lambda b, r: (b, 0, 0)),
        ],
        out_specs=pl.BlockSpec((1, RB2, 1), lambda b, r: (b, r, 0)),
        out_shape=jax.ShapeDtypeStruct((B, N, 1), jnp.int32),
    )(dis, dis.reshape(B, 1, N))

    full = lambda s: pl.BlockSpec(s, lambda b, r: (0, 0))
    logit = pl.pallas_call(
        functools.partial(_main_kernel, Np, k_close, k_far, 22),
        grid=(B, N // RB),
        in_specs=[
            pl.BlockSpec((1, RB, 3), lambda b, r: (b, r, 0)),
            pl.BlockSpec((1, 3, Np), lambda b, r: (b, 0, 0)),
            pl.BlockSpec((1, Np, C), lambda b, r: (b, 0, 0)),
            pl.BlockSpec((1, RB, 1), lambda b, r: (b, r, 0)),
            full(W_ind_c.shape), full((1, b_ind_c.shape[0])),
            full(W1_c.shape), full((1, b1_c.shape[0])),
            full(W2_c.shape), full((1, b2_c.shape[0])),
            full(W_ind_f.shape), full((1, b_ind_f.shape[0])),
            full(W1_f.shape), full((1, b1_f.shape[0])),
            full(W2_f.shape), full((1, b2_f.shape[0])),
        ],
        out_specs=pl.BlockSpec((1, RB, 2), lambda b, r: (b, r, 0)),
        out_shape=jax.ShapeDtypeStruct((B, N, 2), jnp.float32),
        scratch_shapes=[
            pltpu.VMEM((RB, Np), jnp.int32),
        ],
    )(detect_point, pct, feature5, is_close,
      W_ind_c, b2(b_ind_c), W1_c, b2(b1_c), W2_c, b2(b2_c),
      W_ind_f, b2(b_ind_f), W1_f, b2(b1_f), W2_f, b2(b2_f))
    return logit
